# initial kernel scaffold (unmeasured)
import jax
import jax.numpy as jnp
from jax import lax
from jax.experimental import pallas as pl
from jax.experimental.pallas import tpu as pltpu

N_DEV = 4
M = 8192
D = 2048
CHUNK = M // N_DEV


def kernel(partial, gamma):
    partial = partial.reshape(M, D)
    gamma2d = gamma.reshape(1, D)

    def body(partial_ref, gamma_ref, out_ref, comm_ref, local_ref,
             send_sems, recv_sems, local_sem):
        my_x = lax.axis_index("x")
        my_y = lax.axis_index("y")
        my_z = lax.axis_index("z")
        left = (my_z - 1) % N_DEV
        right = (my_z + 1) % N_DEV

        barrier_sem = pltpu.get_barrier_semaphore()
        for nbr in (left, right):
            pl.semaphore_signal(
                barrier_sem, inc=1,
                device_id=(my_x, my_y, nbr),
                device_id_type=pl.DeviceIdType.MESH,
            )
        pl.semaphore_wait(barrier_sem, 2)

        c0 = (my_z - 1) % N_DEV
        rdma0 = pltpu.make_async_remote_copy(
            src_ref=partial_ref.at[pl.ds(c0 * CHUNK, CHUNK), :],
            dst_ref=comm_ref.at[0],
            send_sem=send_sems.at[0],
            recv_sem=recv_sems.at[0],
            device_id=(my_x, my_y, right),
            device_id_type=pl.DeviceIdType.MESH,
        )
        rdma0.start()

        r0 = (my_z - 2) % N_DEV
        cp0 = pltpu.make_async_copy(
            partial_ref.at[pl.ds(r0 * CHUNK, CHUNK), :], local_ref, local_sem
        )
        cp0.start()
        rdma0.wait()
        cp0.wait()
        comm_ref[0, :, :] += local_ref[:, :]

        for s in range(1, N_DEV - 1):
            rdma = pltpu.make_async_remote_copy(
                src_ref=comm_ref.at[s - 1],
                dst_ref=comm_ref.at[s],
                send_sem=send_sems.at[s],
                recv_sem=recv_sems.at[s],
                device_id=(my_x, my_y, right),
                device_id_type=pl.DeviceIdType.MESH,
            )
            rdma.start()
            rs = (my_z - s - 2) % N_DEV
            cp = pltpu.make_async_copy(
                partial_ref.at[pl.ds(rs * CHUNK, CHUNK), :], local_ref,
                local_sem,
            )
            cp.start()
            rdma.wait()
            cp.wait()
            last = s == N_DEV - 2
            if not last:
                comm_ref[s, :, :] += local_ref[:, :]
            else:
                y = comm_ref[s, :, :] + local_ref[:, :]
                msq = jnp.mean(y * y, axis=-1, keepdims=True)
                out_ref[:, :] = y * lax.rsqrt(msq + 1e-6) * gamma_ref[:, :]

    return pl.pallas_call(
        body,
        out_shape=jax.ShapeDtypeStruct((CHUNK, D), jnp.float32),
        in_specs=[
            pl.BlockSpec(memory_space=pltpu.ANY),
            pl.BlockSpec(memory_space=pltpu.VMEM),
        ],
        out_specs=pl.BlockSpec(memory_space=pltpu.VMEM),
        scratch_shapes=[
            pltpu.VMEM((N_DEV - 1, CHUNK, D), jnp.float32),
            pltpu.VMEM((CHUNK, D), jnp.float32),
            pltpu.SemaphoreType.DMA((N_DEV - 1,)),
            pltpu.SemaphoreType.DMA((N_DEV - 1,)),
            pltpu.SemaphoreType.DMA,
        ],
        compiler_params=pltpu.CompilerParams(collective_id=0),
    )(partial, gamma2d)


# baseline (device time: 583748 ns/iter reference)
import jax
import jax.numpy as jnp
from jax import lax
from jax.experimental import pallas as pl
from jax.experimental.pallas import tpu as pltpu

N_DEV = 4
M = 8192
D = 2048
CHUNK = M // N_DEV
R = 4
TR = CHUNK // R


def kernel(partial, gamma):
    partial = partial.reshape(M, D)
    gamma2d = gamma.reshape(1, D)

    def body(partial_ref, gamma_ref, out_ref, comm_ref, local_ref,
             send_sems, recv_sems, local_sem):
        my_x = lax.axis_index("x")
        my_y = lax.axis_index("y")
        my_z = lax.axis_index("z")
        left = (my_z - 1) % N_DEV
        right = (my_z + 1) % N_DEV

        barrier_sem = pltpu.get_barrier_semaphore()
        for nbr in (left, right):
            pl.semaphore_signal(
                barrier_sem, inc=1,
                device_id=(my_x, my_y, nbr),
                device_id_type=pl.DeviceIdType.MESH,
            )
        pl.semaphore_wait(barrier_sem, 2)

        for r in range(R):
            for s in range(N_DEV - 1):
                if s == 0:
                    c_send = (my_z - 1) % N_DEV
                    src = partial_ref.at[pl.ds(c_send * CHUNK + r * TR, TR), :]
                else:
                    src = comm_ref.at[s - 1]
                rdma = pltpu.make_async_remote_copy(
                    src_ref=src,
                    dst_ref=comm_ref.at[s],
                    send_sem=send_sems.at[s],
                    recv_sem=recv_sems.at[s],
                    device_id=(my_x, my_y, right),
                    device_id_type=pl.DeviceIdType.MESH,
                )
                rdma.start()
                c_recv = (my_z - s - 2) % N_DEV
                cp = pltpu.make_async_copy(
                    partial_ref.at[pl.ds(c_recv * CHUNK + r * TR, TR), :],
                    local_ref, local_sem,
                )
                cp.start()
                rdma.wait()
                cp.wait()
                if s < N_DEV - 2:
                    comm_ref[s, :, :] += local_ref[:, :]
                else:
                    y = comm_ref[s, :, :] + local_ref[:, :]
                    msq = jnp.mean(y * y, axis=-1, keepdims=True)
                    out_ref[r * TR:(r + 1) * TR, :] = (
                        y * lax.rsqrt(msq + 1e-6) * gamma_ref[:, :]
                    )

    return pl.pallas_call(
        body,
        out_shape=jax.ShapeDtypeStruct((CHUNK, D), jnp.float32),
        in_specs=[
            pl.BlockSpec(memory_space=pl.ANY),
            pl.BlockSpec(memory_space=pltpu.VMEM),
        ],
        out_specs=pl.BlockSpec(memory_space=pltpu.VMEM),
        scratch_shapes=[
            pltpu.VMEM((N_DEV - 1, TR, D), jnp.float32),
            pltpu.VMEM((TR, D), jnp.float32),
            pltpu.SemaphoreType.DMA((N_DEV - 1,)),
            pltpu.SemaphoreType.DMA((N_DEV - 1,)),
            pltpu.SemaphoreType.DMA,
        ],
        compiler_params=pltpu.CompilerParams(collective_id=0),
    )(partial, gamma2d)


# device time: 294187 ns/iter; 1.9843x vs baseline; 1.9843x over previous
import jax
import jax.numpy as jnp
from jax import lax
from jax.experimental import pallas as pl
from jax.experimental.pallas import tpu as pltpu

N_DEV = 4
M = 8192
D = 2048
CHUNK = M // N_DEV
W = D // 4


def kernel(partial, gamma):
    partial = partial.reshape(M, D)
    gamma2d = gamma.reshape(1, D)

    def body(partial_ref, gamma_ref, out_ref, comm_ref, local_ref,
             zsend_sems, zrecv_sems, xsend_sem, xrecv_sem,
             ysend_sem, yrecv_sem, local_sem):
        my_x = lax.axis_index("x")
        my_y = lax.axis_index("y")
        my_z = lax.axis_index("z")
        left = (my_z - 1) % N_DEV
        right = (my_z + 1) % N_DEV
        q = my_x + 2 * my_y

        barrier_sem = pltpu.get_barrier_semaphore()
        for dev in (
            (my_x, my_y, left),
            (my_x, my_y, right),
            (1 - my_x, my_y, my_z),
            (my_x, 1 - my_y, my_z),
        ):
            pl.semaphore_signal(
                barrier_sem, inc=1,
                device_id=dev, device_id_type=pl.DeviceIdType.MESH,
            )
        pl.semaphore_wait(barrier_sem, 4)

        for s in range(N_DEV - 1):
            if s == 0:
                c_send = (my_z - 1) % N_DEV
                src = partial_ref.at[
                    pl.ds(c_send * CHUNK, CHUNK), pl.ds(q * W, W)
                ]
            else:
                src = comm_ref.at[s - 1]
            rdma = pltpu.make_async_remote_copy(
                src_ref=src,
                dst_ref=comm_ref.at[s],
                send_sem=zsend_sems.at[s],
                recv_sem=zrecv_sems.at[s],
                device_id=(my_x, my_y, right),
                device_id_type=pl.DeviceIdType.MESH,
            )
            rdma.start()
            c_recv = (my_z - s - 2) % N_DEV
            cp = pltpu.make_async_copy(
                partial_ref.at[pl.ds(c_recv * CHUNK, CHUNK), pl.ds(q * W, W)],
                local_ref, local_sem,
            )
            cp.start()
            rdma.wait()
            cp.wait()
            if s < N_DEV - 2:
                comm_ref[s, :, :] += local_ref[:, :]
            else:
                out_ref[:, pl.ds(q * W, W)] = comm_ref[s, :, :] + local_ref[:, :]

        xchg = pltpu.make_async_remote_copy(
            src_ref=out_ref.at[:, pl.ds(q * W, W)],
            dst_ref=out_ref.at[:, pl.ds(q * W, W)],
            send_sem=xsend_sem,
            recv_sem=xrecv_sem,
            device_id=(1 - my_x, my_y, my_z),
            device_id_type=pl.DeviceIdType.MESH,
        )
        xchg.start()
        xchg.wait()

        ychg = pltpu.make_async_remote_copy(
            src_ref=out_ref.at[:, pl.ds(my_y * (2 * W), 2 * W)],
            dst_ref=out_ref.at[:, pl.ds(my_y * (2 * W), 2 * W)],
            send_sem=ysend_sem,
            recv_sem=yrecv_sem,
            device_id=(my_x, 1 - my_y, my_z),
            device_id_type=pl.DeviceIdType.MESH,
        )
        ychg.start()
        ychg.wait()

        RP = 4
        TR = CHUNK // RP
        for rp in range(RP):
            y = out_ref[rp * TR:(rp + 1) * TR, :]
            msq = jnp.mean(y * y, axis=-1, keepdims=True)
            out_ref[rp * TR:(rp + 1) * TR, :] = (
                y * lax.rsqrt(msq + 1e-6) * gamma_ref[:, :]
            )

    return pl.pallas_call(
        body,
        out_shape=jax.ShapeDtypeStruct((CHUNK, D), jnp.float32),
        in_specs=[
            pl.BlockSpec(memory_space=pl.ANY),
            pl.BlockSpec(memory_space=pltpu.VMEM),
        ],
        out_specs=pl.BlockSpec(memory_space=pltpu.VMEM),
        scratch_shapes=[
            pltpu.VMEM((N_DEV - 1, CHUNK, W), jnp.float32),
            pltpu.VMEM((CHUNK, W), jnp.float32),
            pltpu.SemaphoreType.DMA((N_DEV - 1,)),
            pltpu.SemaphoreType.DMA((N_DEV - 1,)),
            pltpu.SemaphoreType.DMA,
            pltpu.SemaphoreType.DMA,
            pltpu.SemaphoreType.DMA,
            pltpu.SemaphoreType.DMA,
            pltpu.SemaphoreType.DMA,
        ],
        compiler_params=pltpu.CompilerParams(collective_id=0),
    )(partial, gamma2d)


# device time: 203372 ns/iter; 2.8703x vs baseline; 1.4465x over previous
import jax
import jax.numpy as jnp
from jax import lax
from jax.experimental import pallas as pl
from jax.experimental.pallas import tpu as pltpu

N_DEV = 4
M = 8192
D = 2048
CHUNK = M // N_DEV
W = D // 4
P = 4
TR = CHUNK // P
S = N_DEV - 1

_ORDER = [(0, 0), (1, 0), (0, 1), (1, 1), (0, 2), (1, 2),
          (2, 0), (3, 0), (2, 1), (3, 1), (2, 2), (3, 2)]


def kernel(partial, gamma):
    partial = partial.reshape(M, D)
    gamma2d = gamma.reshape(1, D)

    def body(partial_ref, gamma_ref, out_ref, comm_ref, local_ref,
             zsend, zrecv, xsend, xrecv, ysend, yrecv, local_sem):
        my_x = lax.axis_index("x")
        my_y = lax.axis_index("y")
        my_z = lax.axis_index("z")
        left = (my_z - 1) % N_DEV
        right = (my_z + 1) % N_DEV
        q = my_x + 2 * my_y

        barrier_sem = pltpu.get_barrier_semaphore()
        for dev in (
            (my_x, my_y, left),
            (my_x, my_y, right),
            (1 - my_x, my_y, my_z),
            (my_x, 1 - my_y, my_z),
        ):
            pl.semaphore_signal(
                barrier_sem, inc=1,
                device_id=dev, device_id_type=pl.DeviceIdType.MESH,
            )
        pl.semaphore_wait(barrier_sem, 4)

        def z_rdma(p, s):
            if s == 0:
                c = (my_z - 1) % N_DEV
                src = partial_ref.at[
                    pl.ds(c * CHUNK + p * TR, TR), pl.ds(q * W, W)
                ]
            else:
                src = comm_ref.at[p, s - 1]
            return pltpu.make_async_remote_copy(
                src_ref=src,
                dst_ref=comm_ref.at[p, s],
                send_sem=zsend.at[p, s],
                recv_sem=zrecv.at[p, s],
                device_id=(my_x, my_y, right),
                device_id_type=pl.DeviceIdType.MESH,
            )

        def cp_make(p, s):
            c = (my_z - s - 2) % N_DEV
            return pltpu.make_async_copy(
                partial_ref.at[
                    pl.ds(c * CHUNK + p * TR, TR), pl.ds(q * W, W)
                ],
                local_ref, local_sem,
            )

        def xchg_make(p):
            sl = out_ref.at[pl.ds(p * TR, TR), pl.ds(q * W, W)]
            return pltpu.make_async_remote_copy(
                src_ref=sl, dst_ref=sl,
                send_sem=xsend.at[p], recv_sem=xrecv.at[p],
                device_id=(1 - my_x, my_y, my_z),
                device_id_type=pl.DeviceIdType.MESH,
            )

        def ychg_make(p):
            sl = out_ref.at[pl.ds(p * TR, TR), pl.ds(my_y * (2 * W), 2 * W)]
            return pltpu.make_async_remote_copy(
                src_ref=sl, dst_ref=sl,
                send_sem=ysend.at[p], recv_sem=yrecv.at[p],
                device_id=(my_x, 1 - my_y, my_z),
                device_id_type=pl.DeviceIdType.MESH,
            )

        rdmas, cps, xchgs, ychgs = {}, {}, {}, {}
        rdmas[(0, 0)] = z_rdma(0, 0)
        rdmas[(0, 0)].start()
        cps[(0, 0)] = cp_make(0, 0)
        cps[(0, 0)].start()
        rdmas[(1, 0)] = z_rdma(1, 0)
        rdmas[(1, 0)].start()

        for k, (p, s) in enumerate(_ORDER):
            rdmas[(p, s)].wait()
            cps[(p, s)].wait()
            if s < S - 1:
                comm_ref[p, s] += local_ref[:, :]
                rdmas[(p, s + 1)] = z_rdma(p, s + 1)
                rdmas[(p, s + 1)].start()
            else:
                out_ref[p * TR:(p + 1) * TR, pl.ds(q * W, W)] = (
                    comm_ref[p, s] + local_ref[:, :]
                )
                xchgs[p] = xchg_make(p)
                xchgs[p].start()
            if (p, s) == (0, 1):
                rdmas[(2, 0)] = z_rdma(2, 0)
                rdmas[(2, 0)].start()
            if (p, s) == (1, 1):
                rdmas[(3, 0)] = z_rdma(3, 0)
                rdmas[(3, 0)].start()
            if k + 1 < len(_ORDER):
                pn, sn = _ORDER[k + 1]
                cps[(pn, sn)] = cp_make(pn, sn)
                cps[(pn, sn)].start()
            if (p, s) == (2, 0):
                xchgs[0].wait()
                ychgs[0] = ychg_make(0)
                ychgs[0].start()
            if (p, s) == (3, 0):
                xchgs[1].wait()
                ychgs[1] = ychg_make(1)
                ychgs[1].start()
            if (p, s) == (2, 1):
                ychgs[0].wait()
            if (p, s) == (3, 1):
                ychgs[1].wait()

        xchgs[2].wait()
        ychgs[2] = ychg_make(2)
        ychgs[2].start()
        xchgs[3].wait()
        ychgs[3] = ychg_make(3)
        ychgs[3].start()
        ychgs[2].wait()
        ychgs[3].wait()

        for rp in range(P):
            y = out_ref[rp * TR:(rp + 1) * TR, :]
            msq = jnp.mean(y * y, axis=-1, keepdims=True)
            out_ref[rp * TR:(rp + 1) * TR, :] = (
                y * lax.rsqrt(msq + 1e-6) * gamma_ref[:, :]
            )

    return pl.pallas_call(
        body,
        out_shape=jax.ShapeDtypeStruct((CHUNK, D), jnp.float32),
        in_specs=[
            pl.BlockSpec(memory_space=pl.ANY),
            pl.BlockSpec(memory_space=pltpu.VMEM),
        ],
        out_specs=pl.BlockSpec(memory_space=pltpu.VMEM),
        scratch_shapes=[
            pltpu.VMEM((P, S, TR, W), jnp.float32),
            pltpu.VMEM((TR, W), jnp.float32),
            pltpu.SemaphoreType.DMA((P, S)),
            pltpu.SemaphoreType.DMA((P, S)),
            pltpu.SemaphoreType.DMA((P,)),
            pltpu.SemaphoreType.DMA((P,)),
            pltpu.SemaphoreType.DMA((P,)),
            pltpu.SemaphoreType.DMA((P,)),
            pltpu.SemaphoreType.DMA,
        ],
        compiler_params=pltpu.CompilerParams(collective_id=0),
    )(partial, gamma2d)
